# Initial kernel scaffold; baseline (speedup 1.0000x reference)
import jax
import jax.numpy as jnp
from jax.experimental import pallas as pl


def kernel(hidden_states, gate_w, corr_bias, Wg, Wu, Wd, Sg, Su, Sd):
    raise NotImplementedError("write your pallas kernel here")


# speedup vs baseline: 1.2704x; 1.2704x over previous
import functools

import jax
import jax.numpy as jnp
from jax.experimental import pallas as pl
from jax.experimental.pallas import tpu as pltpu

T = 2048
H = 1024
I = 512
E = 8
NE = 10
LN = 128
NEG = float("-inf")


def _routing(x, gate_w, corr_bias):
    router_logits = x.astype(jnp.float32) @ gate_w
    routing_weights = jax.nn.softmax(router_logits, axis=1)
    scores = routing_weights + corr_bias.squeeze()
    _, selected_experts = jax.lax.top_k(scores, 2)
    w = jnp.take_along_axis(routing_weights, selected_experts, axis=-1)
    w = w / jnp.clip(jnp.sum(w, axis=-1, keepdims=True), 1e-12)
    return router_logits, selected_experts, w


def _dense_body(x_ref, comb_ref, wg_ref, wu_ref, wd_ref, out_ref, *, bt):
    e = pl.program_id(1)
    xb = x_ref[...].astype(jnp.bfloat16)
    g = jnp.dot(xb, wg_ref[0], preferred_element_type=jnp.float32)
    u = jnp.dot(xb, wu_ref[0], preferred_element_type=jnp.float32)
    h = (g * jax.nn.sigmoid(g) * u).astype(jnp.bfloat16)
    y = jnp.dot(h, wd_ref[0], preferred_element_type=jnp.float32)
    lane = jax.lax.broadcasted_iota(jnp.int32, (bt, LN), 1)
    sel = jnp.sum(jnp.where(lane == e, comb_ref[...], 0.0), axis=1, keepdims=True)
    contrib = sel * y

    @pl.when(e == 0)
    def _():
        out_ref[...] = contrib

    @pl.when(e > 0)
    def _():
        out_ref[...] += contrib


def kernel(hidden_states, gate_w, corr_bias, Wg, Wu, Wd, Sg, Su, Sd):
    b, s, h = hidden_states.shape
    x = hidden_states.reshape(T, H)

    router_logits, selected_experts, w = _routing(x, gate_w, corr_bias)
    onehot = jax.nn.one_hot(selected_experts, E, dtype=w.dtype)
    combine8 = jnp.sum(onehot * w[..., None], axis=1)
    combine = jnp.zeros((T, LN), jnp.float32)
    combine = combine.at[:, :E].set(combine8).at[:, E:NE].set(1.0)

    sg2 = Sg.reshape(H, 2, I).transpose(1, 0, 2)
    su2 = Su.reshape(H, 2, I).transpose(1, 0, 2)
    sd2 = Sd.reshape(2, I, H)
    wg_all = jnp.concatenate([Wg, sg2], axis=0).astype(jnp.bfloat16)
    wu_all = jnp.concatenate([Wu, su2], axis=0).astype(jnp.bfloat16)
    wd_all = jnp.concatenate([Wd, sd2], axis=0).astype(jnp.bfloat16)

    BT = 1024
    grid = (T // BT, NE)
    out = pl.pallas_call(
        functools.partial(_dense_body, bt=BT),
        grid=grid,
        in_specs=[
            pl.BlockSpec((BT, H), lambda i, e: (i, 0)),
            pl.BlockSpec((BT, LN), lambda i, e: (i, 0)),
            pl.BlockSpec((1, H, I), lambda i, e: (e, 0, 0)),
            pl.BlockSpec((1, H, I), lambda i, e: (e, 0, 0)),
            pl.BlockSpec((1, I, H), lambda i, e: (e, 0, 0)),
        ],
        out_specs=pl.BlockSpec((BT, H), lambda i, e: (i, 0)),
        out_shape=jax.ShapeDtypeStruct((T, H), jnp.float32),
    )(x, combine, wg_all, wu_all, wd_all)

    return out.reshape(b, s, h), router_logits
